# baseline (device time: 67939 ns/iter reference)
import jax
import jax.numpy as jnp
from jax import lax
from jax.experimental import pallas as pl
from jax.experimental.pallas import tpu as pltpu


def kernel(O, Wo):
    B, S, Hs, D = O.shape
    K = Hs * D
    N = Wo.shape[1]
    S_half = S // 2

    O = O.reshape(B, S, K)

    def body(o_ref, wo_ref, out_ref, send_buf, recv_buf, send_sem, recv_sem):
        my_x = lax.axis_index("x")
        my_y = lax.axis_index("y")
        my_z = lax.axis_index("z")
        partner = (my_x, 1 - my_y, my_z)

        barrier_sem = pltpu.get_barrier_semaphore()
        pl.semaphore_signal(
            barrier_sem, inc=1,
            device_id=partner, device_id_type=pl.DeviceIdType.MESH,
        )
        pl.semaphore_wait(barrier_sem, 1)

        wo = wo_ref[...].astype(jnp.bfloat16)

        part_start = (1 - my_y) * S_half
        for b in range(B):
            o_b = o_ref[b, pl.ds(part_start, S_half), :].astype(jnp.bfloat16)
            send_buf[b, :, :] = jnp.dot(
                o_b, wo, preferred_element_type=jnp.float32
            ).astype(jnp.bfloat16)

        rdma = pltpu.make_async_remote_copy(
            src_ref=send_buf,
            dst_ref=recv_buf,
            send_sem=send_sem,
            recv_sem=recv_sem,
            device_id=partner,
            device_id_type=pl.DeviceIdType.MESH,
        )
        rdma.start()

        my_start = my_y * S_half
        for b in range(B):
            o_b = o_ref[b, pl.ds(my_start, S_half), :].astype(jnp.bfloat16)
            out_ref[b, :, :] = jnp.dot(
                o_b, wo, preferred_element_type=jnp.float32
            )

        rdma.wait_recv()
        for b in range(B):
            out_ref[b, :, :] += recv_buf[b, :, :].astype(jnp.float32)
        rdma.wait_send()

    return pl.pallas_call(
        body,
        out_shape=jax.ShapeDtypeStruct((B, S_half, N), jnp.float32),
        in_specs=[
            pl.BlockSpec(memory_space=pltpu.VMEM),
            pl.BlockSpec(memory_space=pltpu.VMEM),
        ],
        out_specs=pl.BlockSpec(memory_space=pltpu.VMEM),
        scratch_shapes=[
            pltpu.VMEM((B, S_half, N), jnp.bfloat16),
            pltpu.VMEM((B, S_half, N), jnp.bfloat16),
            pltpu.SemaphoreType.DMA,
            pltpu.SemaphoreType.DMA,
        ],
        compiler_params=pltpu.CompilerParams(collective_id=0),
    )(O, Wo)


# device time: 63407 ns/iter; 1.0715x vs baseline; 1.0715x over previous
import jax
import jax.numpy as jnp
from jax import lax
from jax.experimental import pallas as pl
from jax.experimental.pallas import tpu as pltpu


def kernel(O, Wo):
    B, S, Hs, D = O.shape
    K = Hs * D
    N = Wo.shape[1]
    S_half = S // 2

    O = O.reshape(B, S, K)

    CH = 4
    R = S_half // CH
    NCHUNK = B * CH

    def body(o_ref, wo_ref, out_ref, send_buf, recv_buf, send_sems, recv_sems):
        my_x = lax.axis_index("x")
        my_y = lax.axis_index("y")
        my_z = lax.axis_index("z")
        partner = (my_x, 1 - my_y, my_z)

        barrier_sem = pltpu.get_barrier_semaphore()
        pl.semaphore_signal(
            barrier_sem, inc=1,
            device_id=partner, device_id_type=pl.DeviceIdType.MESH,
        )
        pl.semaphore_wait(barrier_sem, 1)

        wo = wo_ref[...].astype(jnp.bfloat16)

        part_start = (1 - my_y) * S_half
        rdmas = []
        for b in range(B):
            for c in range(CH):
                idx = b * CH + c
                o_b = o_ref[b, pl.ds(part_start + c * R, R), :].astype(
                    jnp.bfloat16
                )
                send_buf[b, c * R:(c + 1) * R, :] = jnp.dot(
                    o_b, wo, preferred_element_type=jnp.float32
                ).astype(jnp.bfloat16)
                rdma = pltpu.make_async_remote_copy(
                    src_ref=send_buf.at[b, c * R:(c + 1) * R, :],
                    dst_ref=recv_buf.at[b, c * R:(c + 1) * R, :],
                    send_sem=send_sems.at[idx],
                    recv_sem=recv_sems.at[idx],
                    device_id=partner,
                    device_id_type=pl.DeviceIdType.MESH,
                )
                rdma.start()
                rdmas.append(rdma)

        my_start = my_y * S_half
        for b in range(B):
            o_b = o_ref[b, pl.ds(my_start, S_half), :].astype(jnp.bfloat16)
            out_ref[b, :, :] = jnp.dot(
                o_b, wo, preferred_element_type=jnp.float32
            )

        for b in range(B):
            for c in range(CH):
                rdmas[b * CH + c].wait_recv()
                out_ref[b, c * R:(c + 1) * R, :] += recv_buf[
                    b, c * R:(c + 1) * R, :
                ].astype(jnp.float32)
        for rdma in rdmas:
            rdma.wait_send()

    return pl.pallas_call(
        body,
        out_shape=jax.ShapeDtypeStruct((B, S_half, N), jnp.float32),
        in_specs=[
            pl.BlockSpec(memory_space=pltpu.VMEM),
            pl.BlockSpec(memory_space=pltpu.VMEM),
        ],
        out_specs=pl.BlockSpec(memory_space=pltpu.VMEM),
        scratch_shapes=[
            pltpu.VMEM((B, S_half, N), jnp.bfloat16),
            pltpu.VMEM((B, S_half, N), jnp.bfloat16),
            pltpu.SemaphoreType.DMA((NCHUNK,)),
            pltpu.SemaphoreType.DMA((NCHUNK,)),
        ],
        compiler_params=pltpu.CompilerParams(collective_id=0),
    )(O, Wo)
